# baseline (device time: 80287 ns/iter reference)
import jax
import jax.numpy as jnp
from jax import lax
from jax.experimental import pallas as pl
from jax.experimental.pallas import tpu as pltpu

B = 2
S = 1024
S_HALF = 512
K = 1024
N = 2048
N_HALF = 1024

CHUNK_ROWS = [320, 256, 192, 160, 96]
CHUNK_OFFS = [0, 320, 576, 768, 928]
NCH = len(CHUNK_ROWS)


def _bslices(off, rows):
    out = []
    for b in range(B):
        lo = max(off, b * S_HALF)
        hi = min(off + rows, (b + 1) * S_HALF)
        if lo < hi:
            out.append((b, lo - b * S_HALF, hi - b * S_HALF, lo - off))
    return out


def kernel(O, Wo):
    o_flat = O.reshape(B, S, K)

    def body(o_ref, wo_ref, out_ref, ysend_ref, yrecv_ref, xrecv_ref,
             ysend_sem, yrecv_sem, xsend_sem, xrecv_sem):
        my_x = lax.axis_index("x")
        my_y = lax.axis_index("y")
        other_x = 1 - my_x
        other_y = 1 - my_y

        barrier_sem = pltpu.get_barrier_semaphore()
        for nbr in [(my_x, other_y), (other_x, my_y)]:
            pl.semaphore_signal(
                barrier_sem, inc=1,
                device_id=nbr, device_id_type=pl.DeviceIdType.MESH,
            )
        pl.semaphore_wait(barrier_sem, 2)

        peer_s = other_y * S_HALF
        own_s = my_y * S_HALF
        my_cols = my_x * N_HALF
        ox_cols = other_x * N_HALF

        y_rdmas = []

        def start_y(c):
            rdma = pltpu.make_async_remote_copy(
                src_ref=ysend_ref.at[pl.ds(CHUNK_OFFS[c], CHUNK_ROWS[c]), :],
                dst_ref=yrecv_ref.at[pl.ds(CHUNK_OFFS[c], CHUNK_ROWS[c]), :],
                send_sem=ysend_sem.at[c],
                recv_sem=yrecv_sem.at[c],
                device_id=(my_x, other_y),
                device_id_type=pl.DeviceIdType.MESH,
            )
            rdma.start()
            y_rdmas.append(rdma)

        ysend_ref[pl.ds(0, S_HALF), :] = jnp.dot(
            o_ref[0, pl.ds(peer_s, S_HALF), :],
            wo_ref[:, pl.ds(my_cols, N_HALF)],
            preferred_element_type=jnp.float32,
        )
        start_y(0)
        ysend_ref[pl.ds(S_HALF, S_HALF), :] = jnp.dot(
            o_ref[1, pl.ds(peer_s, S_HALF), :],
            wo_ref[:, pl.ds(my_cols, N_HALF)],
            preferred_element_type=jnp.float32,
        )
        for c in range(1, NCH):
            start_y(c)

        for b in range(B):
            out_ref[b, :, pl.ds(my_cols, N_HALF)] = jnp.dot(
                o_ref[b, pl.ds(own_s, S_HALF), :],
                wo_ref[:, pl.ds(my_cols, N_HALF)],
                preferred_element_type=jnp.float32,
            )

        fwd_rdmas = []
        for c in range(NCH):
            rows = pl.ds(CHUNK_OFFS[c], CHUNK_ROWS[c])
            y_rdmas[c].wait_recv()
            fwd = pltpu.make_async_remote_copy(
                src_ref=yrecv_ref.at[rows, :],
                dst_ref=xrecv_ref.at[rows, :],
                send_sem=xsend_sem.at[c],
                recv_sem=xrecv_sem.at[c],
                device_id=(other_x, my_y),
                device_id_type=pl.DeviceIdType.MESH,
            )
            fwd.start()
            fwd_rdmas.append(fwd)
            for b, s0, s1, r0 in _bslices(CHUNK_OFFS[c], CHUNK_ROWS[c]):
                out_ref[b, s0:s1, pl.ds(my_cols, N_HALF)] = (
                    out_ref[b, s0:s1, pl.ds(my_cols, N_HALF)]
                    + yrecv_ref[CHUNK_OFFS[c] + r0:CHUNK_OFFS[c] + r0 + (s1 - s0), :]
                )
            if c == 0:
                for b in range(B):
                    out_ref[b, :, pl.ds(ox_cols, N_HALF)] = jnp.dot(
                        o_ref[b, pl.ds(own_s, S_HALF), :],
                        wo_ref[:, pl.ds(ox_cols, N_HALF)],
                        preferred_element_type=jnp.float32,
                    )

        for c in range(NCH):
            fwd_rdmas[c].wait_recv()
            for b, s0, s1, r0 in _bslices(CHUNK_OFFS[c], CHUNK_ROWS[c]):
                out_ref[b, s0:s1, pl.ds(ox_cols, N_HALF)] = (
                    out_ref[b, s0:s1, pl.ds(ox_cols, N_HALF)]
                    + xrecv_ref[CHUNK_OFFS[c] + r0:CHUNK_OFFS[c] + r0 + (s1 - s0), :]
                )

        for c in range(NCH):
            y_rdmas[c].wait_send()
            fwd_rdmas[c].wait_send()

    return pl.pallas_call(
        body,
        out_shape=jax.ShapeDtypeStruct((B, S_HALF, N), jnp.float32),
        in_specs=[
            pl.BlockSpec(memory_space=pltpu.VMEM),
            pl.BlockSpec(memory_space=pltpu.VMEM),
        ],
        out_specs=pl.BlockSpec(memory_space=pltpu.VMEM),
        scratch_shapes=[
            pltpu.VMEM((B * S_HALF, N_HALF), jnp.float32),
            pltpu.VMEM((B * S_HALF, N_HALF), jnp.float32),
            pltpu.VMEM((B * S_HALF, N_HALF), jnp.float32),
            pltpu.SemaphoreType.DMA((NCH,)),
            pltpu.SemaphoreType.DMA((NCH,)),
            pltpu.SemaphoreType.DMA((NCH,)),
            pltpu.SemaphoreType.DMA((NCH,)),
        ],
        compiler_params=pltpu.CompilerParams(collective_id=0),
    )(o_flat, Wo)


# device time: 75383 ns/iter; 1.0651x vs baseline; 1.0651x over previous
import jax
import jax.numpy as jnp
from jax import lax
from jax.experimental import pallas as pl
from jax.experimental.pallas import tpu as pltpu

B = 2
S = 1024
S_HALF = 512
K = 1024
N = 2048
N_HALF = 1024

CHUNK_ROWS = [208, 208, 208, 208, 192]
CHUNK_OFFS = [0, 208, 416, 624, 832]
NCH = len(CHUNK_ROWS)


def _bslices(off, rows):
    out = []
    for b in range(B):
        lo = max(off, b * S_HALF)
        hi = min(off + rows, (b + 1) * S_HALF)
        if lo < hi:
            out.append((b, lo - b * S_HALF, hi - b * S_HALF, lo - off))
    return out


def kernel(O, Wo):
    o_flat = O.reshape(B, S, K)

    def body(o_ref, wo_ref, out_ref, ysend_ref, yrecv_ref, xrecv_ref,
             ysend_sem, yrecv_sem, xsend_sem, xrecv_sem):
        my_x = lax.axis_index("x")
        my_y = lax.axis_index("y")
        other_x = 1 - my_x
        other_y = 1 - my_y

        barrier_sem = pltpu.get_barrier_semaphore()
        for nbr in [(my_x, other_y), (other_x, my_y)]:
            pl.semaphore_signal(
                barrier_sem, inc=1,
                device_id=nbr, device_id_type=pl.DeviceIdType.MESH,
            )
        pl.semaphore_wait(barrier_sem, 2)

        peer_s = other_y * S_HALF
        own_s = my_y * S_HALF
        my_cols = my_x * N_HALF
        ox_cols = other_x * N_HALF

        y_rdmas = []

        def start_y(c):
            rdma = pltpu.make_async_remote_copy(
                src_ref=ysend_ref.at[pl.ds(CHUNK_OFFS[c], CHUNK_ROWS[c]), :],
                dst_ref=yrecv_ref.at[pl.ds(CHUNK_OFFS[c], CHUNK_ROWS[c]), :],
                send_sem=ysend_sem.at[c],
                recv_sem=yrecv_sem.at[c],
                device_id=(my_x, other_y),
                device_id_type=pl.DeviceIdType.MESH,
            )
            rdma.start()
            y_rdmas.append(rdma)

        ysend_ref[pl.ds(0, S_HALF), :] = jnp.dot(
            o_ref[0, pl.ds(peer_s, S_HALF), :],
            wo_ref[:, pl.ds(my_cols, N_HALF)],
            preferred_element_type=jnp.float32,
        )
        start_y(0)
        ysend_ref[pl.ds(S_HALF, S_HALF), :] = jnp.dot(
            o_ref[1, pl.ds(peer_s, S_HALF), :],
            wo_ref[:, pl.ds(my_cols, N_HALF)],
            preferred_element_type=jnp.float32,
        )
        for c in range(1, NCH):
            start_y(c)

        for b in range(B):
            out_ref[b, :, pl.ds(my_cols, N_HALF)] = jnp.dot(
                o_ref[b, pl.ds(own_s, S_HALF), :],
                wo_ref[:, pl.ds(my_cols, N_HALF)],
                preferred_element_type=jnp.float32,
            )

        fwd_rdmas = []
        for c in range(NCH):
            rows = pl.ds(CHUNK_OFFS[c], CHUNK_ROWS[c])
            y_rdmas[c].wait_recv()
            fwd = pltpu.make_async_remote_copy(
                src_ref=yrecv_ref.at[rows, :],
                dst_ref=xrecv_ref.at[rows, :],
                send_sem=xsend_sem.at[c],
                recv_sem=xrecv_sem.at[c],
                device_id=(other_x, my_y),
                device_id_type=pl.DeviceIdType.MESH,
            )
            fwd.start()
            fwd_rdmas.append(fwd)
            for b, s0, s1, r0 in _bslices(CHUNK_OFFS[c], CHUNK_ROWS[c]):
                out_ref[b, s0:s1, pl.ds(my_cols, N_HALF)] = (
                    out_ref[b, s0:s1, pl.ds(my_cols, N_HALF)]
                    + yrecv_ref[CHUNK_OFFS[c] + r0:CHUNK_OFFS[c] + r0 + (s1 - s0), :]
                )
            if c == 0:
                for b in range(B):
                    out_ref[b, :, pl.ds(ox_cols, N_HALF)] = jnp.dot(
                        o_ref[b, pl.ds(own_s, S_HALF), :],
                        wo_ref[:, pl.ds(ox_cols, N_HALF)],
                        preferred_element_type=jnp.float32,
                    )

        for c in range(NCH):
            fwd_rdmas[c].wait_recv()
            for b, s0, s1, r0 in _bslices(CHUNK_OFFS[c], CHUNK_ROWS[c]):
                out_ref[b, s0:s1, pl.ds(ox_cols, N_HALF)] = (
                    out_ref[b, s0:s1, pl.ds(ox_cols, N_HALF)]
                    + xrecv_ref[CHUNK_OFFS[c] + r0:CHUNK_OFFS[c] + r0 + (s1 - s0), :]
                )

        for c in range(NCH):
            y_rdmas[c].wait_send()
            fwd_rdmas[c].wait_send()

    return pl.pallas_call(
        body,
        out_shape=jax.ShapeDtypeStruct((B, S_HALF, N), jnp.float32),
        in_specs=[
            pl.BlockSpec(memory_space=pltpu.VMEM),
            pl.BlockSpec(memory_space=pltpu.VMEM),
        ],
        out_specs=pl.BlockSpec(memory_space=pltpu.VMEM),
        scratch_shapes=[
            pltpu.VMEM((B * S_HALF, N_HALF), jnp.float32),
            pltpu.VMEM((B * S_HALF, N_HALF), jnp.float32),
            pltpu.VMEM((B * S_HALF, N_HALF), jnp.float32),
            pltpu.SemaphoreType.DMA((NCH,)),
            pltpu.SemaphoreType.DMA((NCH,)),
            pltpu.SemaphoreType.DMA((NCH,)),
            pltpu.SemaphoreType.DMA((NCH,)),
        ],
        compiler_params=pltpu.CompilerParams(collective_id=0),
    )(o_flat, Wo)


# device time: 69683 ns/iter; 1.1522x vs baseline; 1.0818x over previous
import jax
import jax.numpy as jnp
from jax import lax
from jax.experimental import pallas as pl
from jax.experimental.pallas import tpu as pltpu

B = 2
S = 1024
S_HALF = 512
K = 1024
N = 2048
N_HALF = 1024

CHUNK_ROWS = [56] * 8 + [48] * 12
CHUNK_OFFS = [sum(CHUNK_ROWS[:i]) for i in range(len(CHUNK_ROWS))]
NCH = len(CHUNK_ROWS)


def _bslices(off, rows):
    out = []
    for b in range(B):
        lo = max(off, b * S_HALF)
        hi = min(off + rows, (b + 1) * S_HALF)
        if lo < hi:
            out.append((b, lo - b * S_HALF, hi - b * S_HALF, lo - off))
    return out


def kernel(O, Wo):
    o_flat = O.reshape(B, S, K)

    def body(o_ref, wo_ref, out_ref, ysend_ref, yrecv_ref, xrecv_ref,
             ysend_sem, yrecv_sem, xsend_sem, xrecv_sem):
        my_x = lax.axis_index("x")
        my_y = lax.axis_index("y")
        other_x = 1 - my_x
        other_y = 1 - my_y

        barrier_sem = pltpu.get_barrier_semaphore()
        for nbr in [(my_x, other_y), (other_x, my_y)]:
            pl.semaphore_signal(
                barrier_sem, inc=1,
                device_id=nbr, device_id_type=pl.DeviceIdType.MESH,
            )
        pl.semaphore_wait(barrier_sem, 2)

        peer_s = other_y * S_HALF
        own_s = my_y * S_HALF
        my_cols = my_x * N_HALF
        ox_cols = other_x * N_HALF

        y_rdmas = []

        def start_y(c):
            rdma = pltpu.make_async_remote_copy(
                src_ref=ysend_ref.at[pl.ds(CHUNK_OFFS[c], CHUNK_ROWS[c]), :],
                dst_ref=yrecv_ref.at[pl.ds(CHUNK_OFFS[c], CHUNK_ROWS[c]), :],
                send_sem=ysend_sem.at[c],
                recv_sem=yrecv_sem.at[c],
                device_id=(my_x, other_y),
                device_id_type=pl.DeviceIdType.MESH,
            )
            rdma.start()
            y_rdmas.append(rdma)

        ysend_ref[pl.ds(0, S_HALF), :] = jnp.dot(
            o_ref[0, pl.ds(peer_s, S_HALF), :],
            wo_ref[:, pl.ds(my_cols, N_HALF)],
            preferred_element_type=jnp.float32,
        )
        start_y(0)
        ysend_ref[pl.ds(S_HALF, S_HALF), :] = jnp.dot(
            o_ref[1, pl.ds(peer_s, S_HALF), :],
            wo_ref[:, pl.ds(my_cols, N_HALF)],
            preferred_element_type=jnp.float32,
        )
        for c in range(1, NCH):
            start_y(c)

        for b in range(B):
            out_ref[b, :, pl.ds(my_cols, N_HALF)] = jnp.dot(
                o_ref[b, pl.ds(own_s, S_HALF), :],
                wo_ref[:, pl.ds(my_cols, N_HALF)],
                preferred_element_type=jnp.float32,
            )

        fwd_rdmas = []
        for c in range(NCH):
            rows = pl.ds(CHUNK_OFFS[c], CHUNK_ROWS[c])
            y_rdmas[c].wait_recv()
            fwd = pltpu.make_async_remote_copy(
                src_ref=yrecv_ref.at[rows, :],
                dst_ref=xrecv_ref.at[rows, :],
                send_sem=xsend_sem.at[c],
                recv_sem=xrecv_sem.at[c],
                device_id=(other_x, my_y),
                device_id_type=pl.DeviceIdType.MESH,
            )
            fwd.start()
            fwd_rdmas.append(fwd)
            for b, s0, s1, r0 in _bslices(CHUNK_OFFS[c], CHUNK_ROWS[c]):
                out_ref[b, s0:s1, pl.ds(my_cols, N_HALF)] = (
                    out_ref[b, s0:s1, pl.ds(my_cols, N_HALF)]
                    + yrecv_ref[CHUNK_OFFS[c] + r0:CHUNK_OFFS[c] + r0 + (s1 - s0), :]
                )
            if c == 0:
                for b in range(B):
                    out_ref[b, :, pl.ds(ox_cols, N_HALF)] = jnp.dot(
                        o_ref[b, pl.ds(own_s, S_HALF), :],
                        wo_ref[:, pl.ds(ox_cols, N_HALF)],
                        preferred_element_type=jnp.float32,
                    )

        for c in range(NCH):
            fwd_rdmas[c].wait_recv()
            for b, s0, s1, r0 in _bslices(CHUNK_OFFS[c], CHUNK_ROWS[c]):
                out_ref[b, s0:s1, pl.ds(ox_cols, N_HALF)] = (
                    out_ref[b, s0:s1, pl.ds(ox_cols, N_HALF)]
                    + xrecv_ref[CHUNK_OFFS[c] + r0:CHUNK_OFFS[c] + r0 + (s1 - s0), :]
                )

        for c in range(NCH):
            y_rdmas[c].wait_send()
            fwd_rdmas[c].wait_send()

    return pl.pallas_call(
        body,
        out_shape=jax.ShapeDtypeStruct((B, S_HALF, N), jnp.float32),
        in_specs=[
            pl.BlockSpec(memory_space=pltpu.VMEM),
            pl.BlockSpec(memory_space=pltpu.VMEM),
        ],
        out_specs=pl.BlockSpec(memory_space=pltpu.VMEM),
        scratch_shapes=[
            pltpu.VMEM((B * S_HALF, N_HALF), jnp.float32),
            pltpu.VMEM((B * S_HALF, N_HALF), jnp.float32),
            pltpu.VMEM((B * S_HALF, N_HALF), jnp.float32),
            pltpu.SemaphoreType.DMA((NCH,)),
            pltpu.SemaphoreType.DMA((NCH,)),
            pltpu.SemaphoreType.DMA((NCH,)),
            pltpu.SemaphoreType.DMA((NCH,)),
        ],
        compiler_params=pltpu.CompilerParams(collective_id=0),
    )(o_flat, Wo)
